# baseline (speedup 1.0000x reference)
import functools
import math

import jax
import jax.numpy as jnp
from jax import lax
from jax.experimental import pallas as pl
from jax.experimental.pallas import tpu as pltpu
from jax.experimental.pallas import tpu_sc as plsc

NB = 32
L = 16
CHUNK = 8000
NW = 32

_LN2 = 0.6931471805599453
_SQRT2 = 1.4142135623730951
_TPISQ = -2.0 * math.pi * math.pi


def _ln_ratio(fo, fc):
    a = jnp.maximum(fo, 0.001)
    b = jnp.maximum(jnp.abs(fc), 0.001)
    ba = plsc.bitcast(a, jnp.int32)
    bb = plsc.bitcast(b, jnp.int32)
    e = ((ba >> 23) - (bb >> 23)).astype(jnp.float32)
    ma = plsc.bitcast((ba & 0x007FFFFF) | 0x3F800000, jnp.float32)
    mb = plsc.bitcast((bb & 0x007FFFFF) | 0x3F800000, jnp.float32)
    z = (ma - mb) / (ma + mb)
    z2 = z * z
    p = 1.0 + z2 * (1.0 / 3.0 + z2 * (0.2 + z2 * (1.0 / 7.0 + z2 * (1.0 / 9.0))))
    return e * _LN2 + 2.0 * z * p


def _nchunks_for(wid, nchunks):
    return (nchunks // NW) + jnp.where(wid < (nchunks % NW), 1, 0)


def _binstats_body(nchunks, fcalc_h, fobs_h, bins_h, sums_h, cnts_h,
                   fc_v, fo_v, bi_v, acc_s, acc_c):
    wid = lax.axis_index("s") * 2 + lax.axis_index("c")
    z16 = jnp.zeros((L,), jnp.float32)
    acc_s[pl.ds(0, L)] = z16
    acc_s[pl.ds(L, L)] = z16
    acc_c[pl.ds(0, L)] = z16
    acc_c[pl.ds(L, L)] = z16
    nw = _nchunks_for(wid, nchunks)

    def chunk_body(k, _):
        base = (wid + NW * k) * CHUNK
        pltpu.sync_copy(fcalc_h.at[pl.ds(base, CHUNK)], fc_v)
        pltpu.sync_copy(fobs_h.at[pl.ds(base, CHUNK)], fo_v)
        pltpu.sync_copy(bins_h.at[pl.ds(base, CHUNK)], bi_v)
        b0 = bi_v[pl.ds(0, L)][0]
        b1 = bi_v[pl.ds(CHUNK - L, L)][L - 1]

        @pl.when(b0 == b1)
        def _single_bin():
            @plsc.parallel_loop(0, CHUNK // L, 1, unroll=4, carry=z16)
            def vbody(i, acc):
                sl = pl.ds(i * L, L)
                return acc + _ln_ratio(fo_v[sl], fc_v[sl])
            tot = jnp.sum(vbody)
            iot = lax.iota(jnp.int32, L)
            mlo = iot == b0
            mhi = (iot + L) == b0
            acc_s[pl.ds(0, L)] = acc_s[pl.ds(0, L)] + jnp.where(mlo, tot, 0.0)
            acc_s[pl.ds(L, L)] = acc_s[pl.ds(L, L)] + jnp.where(mhi, tot, 0.0)
            cf = jnp.float32(CHUNK)
            acc_c[pl.ds(0, L)] = acc_c[pl.ds(0, L)] + jnp.where(mlo, cf, 0.0)
            acc_c[pl.ds(L, L)] = acc_c[pl.ds(L, L)] + jnp.where(mhi, cf, 0.0)

        @pl.when(b0 != b1)
        def _multi_bin():
            ones = jnp.full((L,), 1.0, jnp.float32)

            def vbody(i, c):
                sl = pl.ds(i * L, L)
                lr = _ln_ratio(fo_v[sl], fc_v[sl])
                b = bi_v[sl]
                plsc.addupdate_scatter(acc_s, [b], lr)
                plsc.addupdate_scatter(acc_c, [b], ones)
                return c
            lax.fori_loop(0, CHUNK // L, vbody, 0)
        return _

    lax.fori_loop(0, nw, chunk_body, 0)
    pltpu.sync_copy(acc_s, sums_h.at[wid])
    pltpu.sync_copy(acc_c, cnts_h.at[wid])


def _apply_body(nchunks, fcalc_h, bins_h, sx_h, sy_h, sz_h, u_h,
                sums_h, cnts_h, out_h,
                fc_v, bi_v, sx_v, sy_v, sz_v, o_v, u_v, sums_v, cnts_v, ls_v):
    wid = lax.axis_index("s") * 2 + lax.axis_index("c")
    z16 = jnp.zeros((L,), jnp.float32)

    pltpu.sync_copy(sums_h, sums_v)
    pltpu.sync_copy(cnts_h, cnts_v)
    pltpu.sync_copy(u_h, u_v)
    s0 = z16
    s1 = z16
    c0 = z16
    c1 = z16
    for r in range(NW):
        s0 = s0 + sums_v[r, pl.ds(0, L)]
        s1 = s1 + sums_v[r, pl.ds(L, L)]
        c0 = c0 + cnts_v[r, pl.ds(0, L)]
        c1 = c1 + cnts_v[r, pl.ds(L, L)]
    ls_v[pl.ds(0, L)] = s0 / (c0 + 1e-6)
    ls_v[pl.ds(L, L)] = s1 / (c1 + 1e-6)

    uvec = u_v[pl.ds(0, L)]
    cxx = _TPISQ * uvec[0]
    cyy = _TPISQ * uvec[1]
    czz = _TPISQ * uvec[2]
    cxy = 2.0 * _TPISQ * uvec[3]
    cxz = 2.0 * _TPISQ * uvec[4]
    cyz = 2.0 * _TPISQ * uvec[5]

    nw = _nchunks_for(wid, nchunks)

    def chunk_body(k, _):
        base = (wid + NW * k) * CHUNK
        pltpu.sync_copy(fcalc_h.at[pl.ds(base, CHUNK)], fc_v)
        pltpu.sync_copy(bins_h.at[pl.ds(base, CHUNK)], bi_v)
        pltpu.sync_copy(sx_h.at[pl.ds(base, CHUNK)], sx_v)
        pltpu.sync_copy(sy_h.at[pl.ds(base, CHUNK)], sy_v)
        pltpu.sync_copy(sz_h.at[pl.ds(base, CHUNK)], sz_v)

        @plsc.parallel_loop(0, CHUNK // L, 1, unroll=4)
        def vbody(i):
            sl = pl.ds(i * L, L)
            fca = jnp.maximum(jnp.abs(fc_v[sl]), 0.001)
            b = bi_v[sl]
            ls = plsc.load_gather(ls_v, [b])
            sx = sx_v[sl]
            sy = sy_v[sl]
            sz = sz_v[sl]
            expo = (cxx * sx * sx + cyy * sy * sy + czz * sz * sz
                    + cxy * sx * sy + cxz * sx * sz + cyz * sy * sz)
            o_v[sl] = fca * jnp.exp(ls + expo)
        pltpu.sync_copy(o_v, out_h.at[pl.ds(base, CHUNK)])
        return _

    lax.fori_loop(0, nw, chunk_body, 0)


def kernel(fcalc, fobs, s, U, bins):
    n = fcalc.shape[0]
    assert n % CHUNK == 0, n
    nchunks = n // CHUNK
    bins32 = bins.astype(jnp.int32)
    sx = s[:, 0]
    sy = s[:, 1]
    sz = s[:, 2]
    u16 = jnp.pad(U.astype(jnp.float32), (0, L - U.shape[0]))
    mesh = plsc.VectorSubcoreMesh(core_axis_name="c", subcore_axis_name="s",
                                  num_cores=2, num_subcores=16)
    f32 = jnp.float32

    kA = pl.kernel(
        functools.partial(_binstats_body, nchunks),
        out_type=(jax.ShapeDtypeStruct((NW, NB), f32),
                  jax.ShapeDtypeStruct((NW, NB), f32)),
        mesh=mesh,
        compiler_params=pltpu.CompilerParams(needs_layout_passes=False),
        scratch_types=[
            pltpu.VMEM((CHUNK,), f32),
            pltpu.VMEM((CHUNK,), f32),
            pltpu.VMEM((CHUNK,), jnp.int32),
            pltpu.VMEM((NB,), f32),
            pltpu.VMEM((NB,), f32),
        ],
    )
    sums, cnts = kA(fcalc, fobs, bins32)

    kB = pl.kernel(
        functools.partial(_apply_body, nchunks),
        out_type=jax.ShapeDtypeStruct((n,), f32),
        mesh=mesh,
        compiler_params=pltpu.CompilerParams(needs_layout_passes=False),
        scratch_types=[
            pltpu.VMEM((CHUNK,), f32),
            pltpu.VMEM((CHUNK,), jnp.int32),
            pltpu.VMEM((CHUNK,), f32),
            pltpu.VMEM((CHUNK,), f32),
            pltpu.VMEM((CHUNK,), f32),
            pltpu.VMEM((CHUNK,), f32),
            pltpu.VMEM((L,), f32),
            pltpu.VMEM((NW, NB), f32),
            pltpu.VMEM((NW, NB), f32),
            pltpu.VMEM((NB,), f32),
        ],
    )
    return kB(fcalc, bins32, sx, sy, sz, u16, sums, cnts)


# speedup vs baseline: 1.0102x; 1.0102x over previous
import functools
import math

import jax
import jax.numpy as jnp
from jax import lax
from jax.experimental import pallas as pl
from jax.experimental.pallas import tpu as pltpu
from jax.experimental.pallas import tpu_sc as plsc

NB = 32
L = 16
CHUNK = 8000
NW = 32

_LN2 = 0.6931471805599453
_SQRT2 = 1.4142135623730951
_TPISQ = -2.0 * math.pi * math.pi


def _ln_ratio(fo, fc):
    a = jnp.maximum(fo, 0.001)
    b = jnp.maximum(jnp.abs(fc), 0.001)
    ba = plsc.bitcast(a, jnp.int32)
    bb = plsc.bitcast(b, jnp.int32)
    e = ((ba >> 23) - (bb >> 23)).astype(jnp.float32)
    ma = plsc.bitcast((ba & 0x007FFFFF) | 0x3F800000, jnp.float32)
    mb = plsc.bitcast((bb & 0x007FFFFF) | 0x3F800000, jnp.float32)
    z = (ma - mb) / (ma + mb)
    z2 = z * z
    p = 1.0 + z2 * (1.0 / 3.0 + z2 * (0.2 + z2 * (1.0 / 7.0 + z2 * (1.0 / 9.0))))
    return e * _LN2 + 2.0 * z * p


def _nchunks_for(wid, nchunks):
    return (nchunks // NW) + jnp.where(wid < (nchunks % NW), 1, 0)


def _binstats_body(nchunks, fcalc_h, fobs_h, bins_h, sums_h, cnts_h,
                   fc_v, fo_v, bi_v, acc_s, acc_c):
    wid = lax.axis_index("s") * 2 + lax.axis_index("c")
    z16 = jnp.zeros((L,), jnp.float32)
    acc_s[pl.ds(0, L)] = z16
    acc_s[pl.ds(L, L)] = z16
    acc_c[pl.ds(0, L)] = z16
    acc_c[pl.ds(L, L)] = z16
    nw = _nchunks_for(wid, nchunks)

    def chunk_body(k, _):
        base = (wid + NW * k) * CHUNK
        pltpu.sync_copy(fcalc_h.at[pl.ds(base, CHUNK)], fc_v)
        pltpu.sync_copy(fobs_h.at[pl.ds(base, CHUNK)], fo_v)
        pltpu.sync_copy(bins_h.at[pl.ds(base, CHUNK)], bi_v)
        b0 = bi_v[pl.ds(0, L)][0]
        b1 = bi_v[pl.ds(CHUNK - L, L)][L - 1]

        @pl.when(b0 == b1)
        def _single_bin():
            @plsc.parallel_loop(0, CHUNK // L, 1, unroll=4, carry=z16)
            def vbody(i, acc):
                sl = pl.ds(i * L, L)
                return acc + _ln_ratio(fo_v[sl], fc_v[sl])
            tot = jnp.sum(vbody)
            iot = lax.iota(jnp.int32, L)
            mlo = iot == b0
            mhi = (iot + L) == b0
            acc_s[pl.ds(0, L)] = acc_s[pl.ds(0, L)] + jnp.where(mlo, tot, 0.0)
            acc_s[pl.ds(L, L)] = acc_s[pl.ds(L, L)] + jnp.where(mhi, tot, 0.0)
            cf = jnp.float32(CHUNK)
            acc_c[pl.ds(0, L)] = acc_c[pl.ds(0, L)] + jnp.where(mlo, cf, 0.0)
            acc_c[pl.ds(L, L)] = acc_c[pl.ds(L, L)] + jnp.where(mhi, cf, 0.0)

        @pl.when(b0 != b1)
        def _multi_bin():
            ones = jnp.full((L,), 1.0, jnp.float32)

            def vbody(i, c):
                sl = pl.ds(i * L, L)
                lr = _ln_ratio(fo_v[sl], fc_v[sl])
                b = bi_v[sl]
                plsc.addupdate_scatter(acc_s, [b], lr)
                plsc.addupdate_scatter(acc_c, [b], ones)
                return c
            lax.fori_loop(0, CHUNK // L, vbody, 0)
        return _

    lax.fori_loop(0, nw, chunk_body, 0)
    pltpu.sync_copy(acc_s, sums_h.at[wid])
    pltpu.sync_copy(acc_c, cnts_h.at[wid])


def _apply_body(nchunks, fcalc_h, bins_h, ex_h,
                sums_h, cnts_h, out_h,
                fc_v, bi_v, ex_v, o_v, sums_v, cnts_v, ls_v):
    wid = lax.axis_index("s") * 2 + lax.axis_index("c")
    z16 = jnp.zeros((L,), jnp.float32)

    pltpu.sync_copy(sums_h, sums_v)
    pltpu.sync_copy(cnts_h, cnts_v)
    s0 = z16
    s1 = z16
    c0 = z16
    c1 = z16
    for r in range(NW):
        s0 = s0 + sums_v[r, pl.ds(0, L)]
        s1 = s1 + sums_v[r, pl.ds(L, L)]
        c0 = c0 + cnts_v[r, pl.ds(0, L)]
        c1 = c1 + cnts_v[r, pl.ds(L, L)]
    ls_v[pl.ds(0, L)] = s0 / (c0 + 1e-6)
    ls_v[pl.ds(L, L)] = s1 / (c1 + 1e-6)

    nw = _nchunks_for(wid, nchunks)

    def chunk_body(k, _):
        base = (wid + NW * k) * CHUNK
        pltpu.sync_copy(fcalc_h.at[pl.ds(base, CHUNK)], fc_v)
        pltpu.sync_copy(bins_h.at[pl.ds(base, CHUNK)], bi_v)
        pltpu.sync_copy(ex_h.at[pl.ds(base, CHUNK)], ex_v)

        @plsc.parallel_loop(0, CHUNK // L, 1, unroll=4)
        def vbody(i):
            sl = pl.ds(i * L, L)
            fca = jnp.maximum(jnp.abs(fc_v[sl]), 0.001)
            b = bi_v[sl]
            ls = plsc.load_gather(ls_v, [b])
            o_v[sl] = fca * jnp.exp(ls + ex_v[sl])
        pltpu.sync_copy(o_v, out_h.at[pl.ds(base, CHUNK)])
        return _

    lax.fori_loop(0, nw, chunk_body, 0)


def _expo_body(u_ref, sx_ref, sy_ref, sz_ref, ex_ref):
    cxx = _TPISQ * u_ref[0, 0]
    cyy = _TPISQ * u_ref[0, 1]
    czz = _TPISQ * u_ref[0, 2]
    cxy = 2.0 * _TPISQ * u_ref[0, 3]
    cxz = 2.0 * _TPISQ * u_ref[0, 4]
    cyz = 2.0 * _TPISQ * u_ref[0, 5]
    sx = sx_ref[...]
    sy = sy_ref[...]
    sz = sz_ref[...]
    ex_ref[...] = (cxx * sx * sx + cyy * sy * sy + czz * sz * sz
                   + cxy * sx * sy + cxz * sx * sz + cyz * sy * sz)


def kernel(fcalc, fobs, s, U, bins):
    n = fcalc.shape[0]
    assert n % CHUNK == 0, n
    nchunks = n // CHUNK
    bins32 = bins.astype(jnp.int32)
    rows = n // 128
    sx = s[:, 0].reshape(rows, 128)
    sy = s[:, 1].reshape(rows, 128)
    sz = s[:, 2].reshape(rows, 128)
    u8 = jnp.pad(U.astype(jnp.float32), (0, 8 - U.shape[0])).reshape(1, 8)
    mesh = plsc.VectorSubcoreMesh(core_axis_name="c", subcore_axis_name="s",
                                  num_cores=2, num_subcores=16)
    f32 = jnp.float32

    expo = pl.pallas_call(
        _expo_body,
        out_shape=jax.ShapeDtypeStruct((rows, 128), f32),
    )(u8, sx, sy, sz).reshape(n)

    kA = pl.kernel(
        functools.partial(_binstats_body, nchunks),
        out_type=(jax.ShapeDtypeStruct((NW, NB), f32),
                  jax.ShapeDtypeStruct((NW, NB), f32)),
        mesh=mesh,
        compiler_params=pltpu.CompilerParams(needs_layout_passes=False),
        scratch_types=[
            pltpu.VMEM((CHUNK,), f32),
            pltpu.VMEM((CHUNK,), f32),
            pltpu.VMEM((CHUNK,), jnp.int32),
            pltpu.VMEM((NB,), f32),
            pltpu.VMEM((NB,), f32),
        ],
    )
    sums, cnts = kA(fcalc, fobs, bins32)

    kB = pl.kernel(
        functools.partial(_apply_body, nchunks),
        out_type=jax.ShapeDtypeStruct((n,), f32),
        mesh=mesh,
        compiler_params=pltpu.CompilerParams(needs_layout_passes=False),
        scratch_types=[
            pltpu.VMEM((CHUNK,), f32),
            pltpu.VMEM((CHUNK,), jnp.int32),
            pltpu.VMEM((CHUNK,), f32),
            pltpu.VMEM((CHUNK,), f32),
            pltpu.VMEM((NW, NB), f32),
            pltpu.VMEM((NW, NB), f32),
            pltpu.VMEM((NB,), f32),
        ],
    )
    return kB(fcalc, bins32, expo, sums, cnts)
